# baseline (speedup 1.0000x reference)
import jax
import jax.numpy as jnp
from jax.experimental import pallas as pl
from jax.experimental.pallas import tpu as pltpu

N = 200
H = 200
Z = 56

_DN_T = (((1,), (1,)), ((), ()))


def _encode_kernel(adjT_ref, types_ref, Wih_ref, bih_ref, Whh_ref, bhh_ref,
                   Wg_ref, bg_ref, Wm_ref, Wmu_ref, bmu_ref, Wlv_ref, blv_ref,
                   mu_ref, lv_ref, G_ref, GI_ref, M_ref, D_ref):
    GI_ref[...] = (jax.lax.dot_general(types_ref[...], Wih_ref[...], _DN_T)
                   + bih_ref[...])
    G_ref[...] = jnp.zeros((N, H), dtype=jnp.float32)
    M_ref[...] = (adjT_ref[...] == 1.0).astype(jnp.float32)
    sub_iota = jax.lax.broadcasted_iota(jnp.int32, (8, H), 0)
    row_i = jax.lax.broadcasted_iota(jnp.int32, (N, N), 0)
    col_i = jax.lax.broadcasted_iota(jnp.int32, (N, N), 1)
    shifted_eye = (col_i == row_i - 1).astype(jnp.float32)
    ones_col = jnp.ones((N, 1), jnp.float32)
    D_ref[...] = jax.lax.dot_general(M_ref[...] * shifted_eye, ones_col,
                                     (((1,), (0,)), ((), ())))

    def gru_tail(v, agg, h_prev):
        gi = GI_ref[pl.ds(v, 1), :]
        gh = jax.lax.dot_general(agg, Whh_ref[...], _DN_T) + bhh_ref[...]
        r = jax.nn.sigmoid(gi[:, 0:H] + gh[:, 0:H])
        z = jax.nn.sigmoid(gi[:, H:2 * H] + gh[:, H:2 * H])
        n = jnp.tanh(gi[:, 2 * H:3 * H] + r * gh[:, 2 * H:3 * H])
        h_new = (1.0 - z) * n + z * agg
        gate = jax.nn.sigmoid(
            jax.lax.dot_general(h_new, Wg_ref[...], _DN_T) + bg_ref[...])
        mapped = jax.lax.dot_general(h_new, Wm_ref[...], _DN_T)
        return h_new, gate * mapped

    def piped_step(v, s_star, h_prev, Grow_prev, mask_col=None):
        if mask_col is None:
            mask_col = M_ref[pl.ds(v, 1), :].reshape(N, 1)
        P = mask_col * G_ref[...]
        acc = P[0:8, :]
        for k in range(1, 25):
            acc = acc + P[8 * k:8 * (k + 1), :]
        G_ref[pl.ds(v - 1, 1), :] = Grow_prev
        c_val = D_ref[pl.ds(v, 1), :]
        c_row = c_val * Grow_prev
        acc = jnp.where(sub_iota == s_star,
                        acc + jnp.broadcast_to(c_row, (8, H)), acc)
        agg = jnp.sum(acc, axis=0, keepdims=True)
        return gru_tail(v, agg, h_prev)

    h, Grow = gru_tail(0, jnp.zeros((1, H), jnp.float32),
                       jnp.zeros((1, H), jnp.float32))
    for v in range(1, 8):
        h, Grow = piped_step(v, v - 1, h, Grow)

    def block(b, carry):
        h, Grow = carry
        MB = jnp.swapaxes(M_ref[pl.ds(8 + 16 * b, 16), :], 0, 1)
        for i in range(16):
            h, Grow = piped_step(8 + 16 * b + i, (i - 1) % 8, h, Grow,
                                 MB[:, i:i + 1])
        return h, Grow

    hg, _ = jax.lax.fori_loop(0, 12, block, (h, Grow))
    mu_ref[...] = jax.lax.dot_general(hg, Wmu_ref[...], _DN_T) + bmu_ref[...]
    lv_ref[...] = jax.lax.dot_general(hg, Wlv_ref[...], _DN_T) + blv_ref[...]


@jax.jit
def kernel(g_in, W_ih, b_ih, W_hh, b_hh, Wg, bg, Wm, W_mu, b_mu, W_lv, b_lv):
    adjT = g_in[0].T
    types = g_in[1]
    mu, lv = pl.pallas_call(
        _encode_kernel,
        out_shape=[jax.ShapeDtypeStruct((1, Z), jnp.float32),
                   jax.ShapeDtypeStruct((1, Z), jnp.float32)],
        scratch_shapes=[pltpu.VMEM((N, H), jnp.float32),
                        pltpu.VMEM((N, 3 * H), jnp.float32),
                        pltpu.VMEM((N, N), jnp.float32),
                        pltpu.VMEM((N, 1), jnp.float32)],
    )(adjT, types, W_ih, b_ih.reshape(1, 3 * H), W_hh, b_hh.reshape(1, 3 * H),
      Wg, bg.reshape(1, H), Wm, W_mu, b_mu.reshape(1, Z), W_lv,
      b_lv.reshape(1, Z))
    return (mu, lv)


# speedup vs baseline: 1.0187x; 1.0187x over previous
import jax
import jax.numpy as jnp
from jax.experimental import pallas as pl
from jax.experimental.pallas import tpu as pltpu

N = 200
H = 200
Z = 56

_DN_T = (((1,), (1,)), ((), ()))


def _encode_kernel(adjT_ref, types_ref, Wih_ref, bih_ref, Whh_ref, bhh_ref,
                   Wg_ref, bg_ref, Wm_ref, Wmu_ref, bmu_ref, Wlv_ref, blv_ref,
                   mu_ref, lv_ref, G_ref, GI_ref, M_ref, D_ref):
    GI_ref[...] = (jax.lax.dot_general(types_ref[...], Wih_ref[...], _DN_T)
                   + bih_ref[...])
    G_ref[...] = jnp.zeros((N, H), dtype=jnp.float32)
    M_ref[...] = (adjT_ref[...] == 1.0).astype(jnp.float32)
    sub_iota = jax.lax.broadcasted_iota(jnp.int32, (8, H), 0)
    row_i = jax.lax.broadcasted_iota(jnp.int32, (N, N), 0)
    col_i = jax.lax.broadcasted_iota(jnp.int32, (N, N), 1)
    shifted_eye = (col_i == row_i - 1).astype(jnp.float32)
    ones_col = jnp.ones((N, 1), jnp.float32)
    D_ref[...] = jax.lax.dot_general(M_ref[...] * shifted_eye, ones_col,
                                     (((1,), (0,)), ((), ())))

    def gru_tail(v, agg, h_prev):
        gi = GI_ref[pl.ds(v, 1), :]
        gh = jax.lax.dot_general(agg, Whh_ref[...], _DN_T) + bhh_ref[...]
        r = jax.nn.sigmoid(gi[:, 0:H] + gh[:, 0:H])
        z = jax.nn.sigmoid(gi[:, H:2 * H] + gh[:, H:2 * H])
        n = jnp.tanh(gi[:, 2 * H:3 * H] + r * gh[:, 2 * H:3 * H])
        h_new = (1.0 - z) * n + z * agg
        gate = jax.nn.sigmoid(
            jax.lax.dot_general(h_new, Wg_ref[...], _DN_T) + bg_ref[...])
        mapped = jax.lax.dot_general(h_new, Wm_ref[...], _DN_T)
        return h_new, gate * mapped

    def piped_step(v, s_star, h_prev, Grow_prev):
        mask = M_ref[pl.ds(v, 1), :]
        P = mask.reshape(N, 1) * G_ref[...]
        acc = P[0:8, :]
        for k in range(1, 25):
            acc = acc + P[8 * k:8 * (k + 1), :]
        G_ref[pl.ds(v - 1, 1), :] = Grow_prev
        c_val = D_ref[pl.ds(v, 1), :]
        c_row = c_val * Grow_prev
        acc = jnp.where(sub_iota == s_star,
                        acc + jnp.broadcast_to(c_row, (8, H)), acc)
        agg = jnp.sum(acc, axis=0, keepdims=True)
        return gru_tail(v, agg, h_prev)

    h, Grow = gru_tail(0, jnp.zeros((1, H), jnp.float32),
                       jnp.zeros((1, H), jnp.float32))
    for v in range(1, 8):
        h, Grow = piped_step(v, v - 1, h, Grow)

    def block(b, carry):
        h, Grow = carry
        for i in range(16):
            h, Grow = piped_step(8 + 16 * b + i, (i - 1) % 8, h, Grow)
        return h, Grow

    hg, _ = jax.lax.fori_loop(0, 12, block, (h, Grow))
    mu_ref[...] = jax.lax.dot_general(hg, Wmu_ref[...], _DN_T) + bmu_ref[...]
    lv_ref[...] = jax.lax.dot_general(hg, Wlv_ref[...], _DN_T) + blv_ref[...]


@jax.jit
def kernel(g_in, W_ih, b_ih, W_hh, b_hh, Wg, bg, Wm, W_mu, b_mu, W_lv, b_lv):
    adjT = g_in[0].T
    types = g_in[1]
    mu, lv = pl.pallas_call(
        _encode_kernel,
        out_shape=[jax.ShapeDtypeStruct((1, Z), jnp.float32),
                   jax.ShapeDtypeStruct((1, Z), jnp.float32)],
        scratch_shapes=[pltpu.VMEM((N, H), jnp.float32),
                        pltpu.VMEM((N, 3 * H), jnp.float32),
                        pltpu.VMEM((N, N), jnp.float32),
                        pltpu.VMEM((N, 1), jnp.float32)],
    )(adjT, types, W_ih, b_ih.reshape(1, 3 * H), W_hh, b_hh.reshape(1, 3 * H),
      Wg, bg.reshape(1, H), Wm, W_mu, b_mu.reshape(1, Z), W_lv,
      b_lv.reshape(1, Z))
    return (mu, lv)
